# Initial kernel scaffold; baseline (speedup 1.0000x reference)
import jax
import jax.numpy as jnp
from jax.experimental import pallas as pl


def kernel(hn, he, edge_vec, emb, norm, fc1_w1, fc1_w2, fc2_w1, fc2_w2, lin1_w, lin2_w, edge_index):
    raise NotImplementedError("write your pallas kernel here")


# speedup vs baseline: 2.2426x; 2.2426x over previous
import functools
import math

import jax
import jax.numpy as jnp
import numpy as np
from jax import lax
from jax.experimental import pallas as pl
from jax.experimental.pallas import tpu as pltpu
from jax.experimental.pallas import tpu_sc as plsc

N = 10000
E = 160000
CM = 16

NC = 2
NS = 16
NW = NC * NS
CH = 128
NCH = 40
EPAD = NW * NCH * CH
RPS = N // NS


def _sc_gather_body(hn_hbm, src_hbm, dst_hbm, hsrc_hbm, hdst_hbm,
                    idx_v, rows_v, sem):
    cid = lax.axis_index("c")
    sid = lax.axis_index("s")
    wid = sid * NC + cid
    for idx_hbm, out_hbm in ((src_hbm, hsrc_hbm), (dst_hbm, hdst_hbm)):
        pltpu.sync_copy(idx_hbm.at[wid], idx_v)

        def _group(g, carry):
            descs = []
            for b in range(4):
                j = g * 4 + b
                descs.append(pltpu.async_copy(
                    hn_hbm.at[idx_v.at[j]], rows_v.at[j], sem))
            for d in descs:
                d.wait()
            return carry

        lax.fori_loop(0, NCH // 4, _group, 0)
        pltpu.sync_copy(rows_v, out_hbm.at[wid])


def _sc_scatter_body(val_hbm, dsti_hbm, out_hbm, idx_v, val_v, zero_v,
                     shared, sem):
    cid = lax.axis_index("c")
    sid = lax.axis_index("s")
    wid = sid * NC + cid
    d1 = pltpu.async_copy(dsti_hbm.at[wid], idx_v, sem)
    d2 = pltpu.async_copy(val_hbm.at[wid], val_v, sem)

    def _zrow(i, carry):
        zero_v[i, :] = jnp.zeros((CM,), jnp.float32)
        return carry

    lax.fori_loop(0, RPS, _zrow, 0)
    pltpu.sync_copy(zero_v, shared.at[pl.ds(sid * RPS, RPS)])
    d1.wait()
    d2.wait()
    plsc.subcore_barrier()

    def _chunk(j, carry):
        pltpu.sync_copy(val_v.at[j], shared.at[idx_v.at[j]], add=True)
        return carry

    lax.fori_loop(0, NCH, _chunk, 0)
    plsc.subcore_barrier()
    pltpu.sync_copy(shared.at[pl.ds(sid * RPS, RPS)],
                    out_hbm.at[cid, pl.ds(sid * RPS, RPS)])


@functools.lru_cache(maxsize=None)
def _sc_kernels():
    mesh = plsc.VectorSubcoreMesh(core_axis_name="c", subcore_axis_name="s",
                                  num_cores=NC, num_subcores=NS)
    gather = pl.kernel(
        _sc_gather_body,
        out_type=(jax.ShapeDtypeStruct((NW, NCH, CH, CM), jnp.float32),
                  jax.ShapeDtypeStruct((NW, NCH, CH, CM), jnp.float32)),
        mesh=mesh,
        scratch_types=[
            pltpu.VMEM((NCH, CH), jnp.int32),
            pltpu.VMEM((NCH, CH, CM), jnp.float32),
            pltpu.SemaphoreType.DMA,
        ],
        compiler_params=pltpu.CompilerParams(use_tc_tiling_on_sc=False),
    )
    scatter = pl.kernel(
        _sc_scatter_body,
        out_type=jax.ShapeDtypeStruct((NC, N, CM), jnp.float32),
        mesh=mesh,
        scratch_types=[
            pltpu.VMEM((NCH, CH), jnp.int32),
            pltpu.VMEM((NCH, CH, CM), jnp.float32),
            pltpu.VMEM((RPS, CM), jnp.float32),
            pltpu.VMEM_SHARED((N, CM), jnp.float32),
            pltpu.SemaphoreType.DMA,
        ],
        compiler_params=pltpu.CompilerParams(use_tc_tiling_on_sc=False),
    )
    return gather, scatter


def _tc_edge_body(he, hs, hd, emb, nrm, w1a, w2a, acat, trep, smat, bcat,
                  he_new, contrib):
    h1 = jnp.maximum(jnp.dot(emb[...], w1a[...],
                             preferred_element_type=jnp.float32), 0.0)
    h2 = jnp.maximum(jnp.dot(emb[...], w2a[...],
                             preferred_element_type=jnp.float32), 0.0)
    feat = jnp.concatenate([he[...], hs[...], hd[...]], axis=1)
    q1 = jnp.dot(feat, acat[...], preferred_element_type=jnp.float32)
    r1 = jnp.dot(h1, trep[...], preferred_element_type=jnp.float32)
    tmp = jnp.maximum(jnp.dot(q1 * r1, smat[...],
                              preferred_element_type=jnp.float32), 0.0)
    q2 = jnp.dot(tmp, bcat[...], preferred_element_type=jnp.float32)
    r2 = jnp.dot(h2, trep[...], preferred_element_type=jnp.float32)
    hen = he[...] + jnp.dot(q2 * r2, smat[...],
                            preferred_element_type=jnp.float32)
    he_new[...] = hen
    contrib[...] = hen * nrm[...]


BE = 2000
GE = E // BE


def _edge_call(he, hs, hd, emb, nrm, w1a, w2a, acat, trep, smat, bcat):
    full = lambda i: (0, 0)
    return pl.pallas_call(
        _tc_edge_body,
        grid=(GE,),
        in_specs=[
            pl.BlockSpec((BE, CM), lambda i: (i, 0)),
            pl.BlockSpec((BE, CM), lambda i: (i, 0)),
            pl.BlockSpec((BE, CM), lambda i: (i, 0)),
            pl.BlockSpec((BE, 10), lambda i: (i, 0)),
            pl.BlockSpec((BE, 1), lambda i: (i, 0)),
            pl.BlockSpec((10, CM), full),
            pl.BlockSpec((10, CM), full),
            pl.BlockSpec((3 * CM, CM * CM), full),
            pl.BlockSpec((CM, CM * CM), full),
            pl.BlockSpec((CM * CM, CM), full),
            pl.BlockSpec((CM, CM * CM), full),
        ],
        out_specs=[
            pl.BlockSpec((BE, CM), lambda i: (i, 0)),
            pl.BlockSpec((BE, CM), lambda i: (i, 0)),
        ],
        out_shape=[
            jax.ShapeDtypeStruct((E, CM), jnp.float32),
            jax.ShapeDtypeStruct((E, CM), jnp.float32),
        ],
        compiler_params=pltpu.CompilerParams(
            dimension_semantics=("parallel",)),
    )(he, hs, hd, emb, nrm, w1a, w2a, acat, trep, smat, bcat)


def _tc_node_body(hn, nf, l1, l2, out):
    ftr = nf[0] + nf[1]
    cat = jnp.concatenate([hn[...], ftr], axis=1)
    h = jnp.maximum(jnp.dot(cat, l1[...],
                            preferred_element_type=jnp.float32), 0.0)
    out[...] = hn[...] + jnp.dot(h, l2[...],
                                 preferred_element_type=jnp.float32)


BN = 2000
GN = N // BN


def _node_call(hn, nf, l1, l2):
    return pl.pallas_call(
        _tc_node_body,
        grid=(GN,),
        in_specs=[
            pl.BlockSpec((BN, CM), lambda i: (i, 0)),
            pl.BlockSpec((NC, BN, CM), lambda i: (0, i, 0)),
            pl.BlockSpec((2 * CM, CM), lambda i: (0, 0)),
            pl.BlockSpec((CM, CM), lambda i: (0, 0)),
        ],
        out_specs=pl.BlockSpec((BN, CM), lambda i: (i, 0)),
        out_shape=jax.ShapeDtypeStruct((N, CM), jnp.float32),
        compiler_params=pltpu.CompilerParams(
            dimension_semantics=("parallel",)),
    )(hn, nf, l1, l2)


def kernel(hn, he, edge_vec, emb, norm, fc1_w1, fc1_w2, fc2_w1, fc2_w2,
           lin1_w, lin2_w, edge_index):
    del edge_vec
    f32 = jnp.float32
    sqrt2 = math.sqrt(2.0)

    w1a = fc1_w1 / math.sqrt(10.0)
    w2a = fc2_w1 / math.sqrt(10.0)
    acat = (fc1_w2.reshape(CM, 3 * CM, CM).transpose(1, 0, 2)
            .reshape(3 * CM, CM * CM)) / (4.0 * math.sqrt(3.0 * CM))
    bcat = (fc2_w2.reshape(CM, CM, CM).transpose(1, 0, 2)
            .reshape(CM, CM * CM)) / 16.0
    trep = jnp.asarray(np.kron(np.eye(CM, dtype=np.float32),
                               np.ones((1, CM), dtype=np.float32)) * sqrt2)
    smat = jnp.asarray(np.kron(np.ones((CM, 1), dtype=np.float32),
                               np.eye(CM, dtype=np.float32)))
    l1 = lin1_w / math.sqrt(2.0 * CM)
    l2 = lin2_w * (sqrt2 / math.sqrt(CM))

    pad = EPAD - E
    src4 = jnp.concatenate(
        [edge_index[0], jnp.zeros((pad,), jnp.int32)]).reshape(NW, NCH, CH)
    dst4 = jnp.concatenate(
        [edge_index[1], jnp.zeros((pad,), jnp.int32)]).reshape(NW, NCH, CH)

    sc_gather, sc_scatter = _sc_kernels()
    hsrc4, hdst4 = sc_gather(hn, src4, dst4)
    hsrc = hsrc4.reshape(EPAD, CM)[:E]
    hdst = hdst4.reshape(EPAD, CM)[:E]

    he_new, contrib = _edge_call(he, hsrc, hdst, emb,
                                 norm.reshape(E, 1).astype(f32),
                                 w1a, w2a, acat, trep, smat, bcat)

    val4 = jnp.concatenate(
        [contrib, jnp.zeros((pad, CM), f32)]).reshape(NW, NCH, CH, CM)
    nf = sc_scatter(val4, dst4)

    hn_new = _node_call(hn, nf, l1, l2)
    return hn_new, he_new


# speedup vs baseline: 2.8955x; 1.2912x over previous
import functools
import math

import jax
import jax.numpy as jnp
import numpy as np
from jax import lax
from jax.experimental import pallas as pl
from jax.experimental.pallas import tpu as pltpu
from jax.experimental.pallas import tpu_sc as plsc

N = 10000
E = 160000
CM = 16

NC = 2
NS = 16
NW = NC * NS
EPW = E // NW
CH = 128
NFULL = EPW // CH
TAIL = EPW - NFULL * CH
RPS = N // NS


def _sc_gather_body(hn_hbm, ei_hbm, hsrc_hbm, hdst_hbm,
                    idx_v, idx8_v, rows_v, rows8_v, sem):
    cid = lax.axis_index("c")
    sid = lax.axis_index("s")
    wid = sid * NC + cid
    base = wid * EPW
    for p, out_hbm in ((0, hsrc_hbm), (1, hdst_hbm)):
        pltpu.sync_copy(ei_hbm.at[p, pl.ds(base, EPW)], idx_v)
        pltpu.sync_copy(ei_hbm.at[p, pl.ds(base + NFULL * CH, TAIL)], idx8_v)

        def _group(g, carry):
            descs = []
            for b in range(3):
                o = (g * 3 + b) * CH
                descs.append(pltpu.async_copy(
                    hn_hbm.at[idx_v.at[pl.ds(o, CH)]],
                    rows_v.at[pl.ds(o, CH)], sem))
            for d in descs:
                d.wait()
            return carry

        lax.fori_loop(0, NFULL // 3, _group, 0)
        pltpu.async_copy(hn_hbm.at[idx8_v], rows8_v, sem).wait()
        pltpu.sync_copy(rows_v, out_hbm.at[pl.ds(base, NFULL * CH)])
        pltpu.sync_copy(rows8_v, out_hbm.at[pl.ds(base + NFULL * CH, TAIL)])


def _sc_scatter_body(val_hbm, ei_hbm, out_hbm, idx2_v, idx8_v, val_v, val8_v,
                     zero_v, shared, sem):
    cid = lax.axis_index("c")
    sid = lax.axis_index("s")
    wid = sid * NC + cid
    base = wid * EPW
    descs = [pltpu.async_copy(val_hbm.at[pl.ds(base, NFULL * CH)], val_v, sem),
             pltpu.async_copy(val_hbm.at[pl.ds(base + NFULL * CH, TAIL)],
                              val8_v, sem),
             pltpu.async_copy(ei_hbm.at[1, pl.ds(base + NFULL * CH, TAIL)],
                              idx8_v, sem)]
    for j in range(NFULL):
        descs.append(pltpu.async_copy(
            ei_hbm.at[1, pl.ds(base + j * CH, CH)], idx2_v.at[j], sem))

    def _zrow(i, carry):
        zero_v[i, :] = jnp.zeros((CM,), jnp.float32)
        return carry

    lax.fori_loop(0, RPS, _zrow, 0)
    pltpu.sync_copy(zero_v, shared.at[pl.ds(sid * RPS, RPS)])
    for d in descs:
        d.wait()
    plsc.subcore_barrier()

    def _chunk(j, carry):
        pltpu.sync_copy(val_v.at[pl.ds(j * CH, CH)],
                        shared.at[idx2_v.at[j]], add=True)
        return carry

    lax.fori_loop(0, NFULL, _chunk, 0)
    pltpu.sync_copy(val8_v, shared.at[idx8_v], add=True)
    plsc.subcore_barrier()
    pltpu.sync_copy(shared.at[pl.ds(sid * RPS, RPS)],
                    out_hbm.at[pl.ds(cid * N + sid * RPS, RPS)])


@functools.lru_cache(maxsize=None)
def _sc_kernels():
    mesh = plsc.VectorSubcoreMesh(core_axis_name="c", subcore_axis_name="s",
                                  num_cores=NC, num_subcores=NS)
    gather = pl.kernel(
        _sc_gather_body,
        out_type=(jax.ShapeDtypeStruct((E, CM), jnp.float32),
                  jax.ShapeDtypeStruct((E, CM), jnp.float32)),
        mesh=mesh,
        scratch_types=[
            pltpu.VMEM((EPW,), jnp.int32),
            pltpu.VMEM((TAIL,), jnp.int32),
            pltpu.VMEM((NFULL * CH, CM), jnp.float32),
            pltpu.VMEM((TAIL, CM), jnp.float32),
            pltpu.SemaphoreType.DMA,
        ],
        compiler_params=pltpu.CompilerParams(use_tc_tiling_on_sc=False),
    )
    scatter = pl.kernel(
        _sc_scatter_body,
        out_type=jax.ShapeDtypeStruct((NC * N, CM), jnp.float32),
        mesh=mesh,
        scratch_types=[
            pltpu.VMEM((NFULL, CH), jnp.int32),
            pltpu.VMEM((TAIL,), jnp.int32),
            pltpu.VMEM((NFULL * CH, CM), jnp.float32),
            pltpu.VMEM((TAIL, CM), jnp.float32),
            pltpu.VMEM((RPS, CM), jnp.float32),
            pltpu.VMEM_SHARED((N, CM), jnp.float32),
            pltpu.SemaphoreType.DMA,
        ],
        compiler_params=pltpu.CompilerParams(use_tc_tiling_on_sc=False),
    )
    return gather, scatter


def _tc_edge_body(he, hs, hd, emb, nrm, w1a, w2a, acat, trep, smat, bcat,
                  he_new, contrib):
    h1 = jnp.maximum(jnp.dot(emb[...], w1a[...],
                             preferred_element_type=jnp.float32), 0.0)
    h2 = jnp.maximum(jnp.dot(emb[...], w2a[...],
                             preferred_element_type=jnp.float32), 0.0)
    feat = jnp.concatenate([he[...], hs[...], hd[...]], axis=1)
    q1 = jnp.dot(feat, acat[...], preferred_element_type=jnp.float32)
    r1 = jnp.dot(h1, trep[...], preferred_element_type=jnp.float32)
    tmp = jnp.maximum(jnp.dot(q1 * r1, smat[...],
                              preferred_element_type=jnp.float32), 0.0)
    q2 = jnp.dot(tmp, bcat[...], preferred_element_type=jnp.float32)
    r2 = jnp.dot(h2, trep[...], preferred_element_type=jnp.float32)
    hen = he[...] + jnp.dot(q2 * r2, smat[...],
                            preferred_element_type=jnp.float32)
    he_new[...] = hen
    contrib[...] = hen * nrm[...]


BE = 2000
GE = E // BE


def _edge_call(he, hs, hd, emb, nrm, w1a, w2a, acat, trep, smat, bcat):
    full = lambda i: (0, 0)
    return pl.pallas_call(
        _tc_edge_body,
        grid=(GE,),
        in_specs=[
            pl.BlockSpec((BE, CM), lambda i: (i, 0)),
            pl.BlockSpec((BE, CM), lambda i: (i, 0)),
            pl.BlockSpec((BE, CM), lambda i: (i, 0)),
            pl.BlockSpec((BE, 10), lambda i: (i, 0)),
            pl.BlockSpec((BE, 1), lambda i: (i, 0)),
            pl.BlockSpec((10, CM), full),
            pl.BlockSpec((10, CM), full),
            pl.BlockSpec((3 * CM, CM * CM), full),
            pl.BlockSpec((CM, CM * CM), full),
            pl.BlockSpec((CM * CM, CM), full),
            pl.BlockSpec((CM, CM * CM), full),
        ],
        out_specs=[
            pl.BlockSpec((BE, CM), lambda i: (i, 0)),
            pl.BlockSpec((BE, CM), lambda i: (i, 0)),
        ],
        out_shape=[
            jax.ShapeDtypeStruct((E, CM), jnp.float32),
            jax.ShapeDtypeStruct((E, CM), jnp.float32),
        ],
        compiler_params=pltpu.CompilerParams(
            dimension_semantics=("parallel",)),
    )(he, hs, hd, emb, nrm, w1a, w2a, acat, trep, smat, bcat)


def _tc_node_body(hn, nf0, nf1, l1, l2, out):
    ftr = nf0[...] + nf1[...]
    cat = jnp.concatenate([hn[...], ftr], axis=1)
    h = jnp.maximum(jnp.dot(cat, l1[...],
                            preferred_element_type=jnp.float32), 0.0)
    out[...] = hn[...] + jnp.dot(h, l2[...],
                                 preferred_element_type=jnp.float32)


BN = 2000
GN = N // BN


def _node_call(hn, nf, l1, l2):
    return pl.pallas_call(
        _tc_node_body,
        grid=(GN,),
        in_specs=[
            pl.BlockSpec((BN, CM), lambda i: (i, 0)),
            pl.BlockSpec((BN, CM), lambda i: (i, 0)),
            pl.BlockSpec((BN, CM), lambda i: (i + GN, 0)),
            pl.BlockSpec((2 * CM, CM), lambda i: (0, 0)),
            pl.BlockSpec((CM, CM), lambda i: (0, 0)),
        ],
        out_specs=pl.BlockSpec((BN, CM), lambda i: (i, 0)),
        out_shape=jax.ShapeDtypeStruct((N, CM), jnp.float32),
        compiler_params=pltpu.CompilerParams(
            dimension_semantics=("parallel",)),
    )(hn, nf, nf, l1, l2)


def kernel(hn, he, edge_vec, emb, norm, fc1_w1, fc1_w2, fc2_w1, fc2_w2,
           lin1_w, lin2_w, edge_index):
    del edge_vec
    f32 = jnp.float32
    sqrt2 = math.sqrt(2.0)

    w1a = fc1_w1 / math.sqrt(10.0)
    w2a = fc2_w1 / math.sqrt(10.0)
    acat = (fc1_w2.reshape(CM, 3 * CM, CM).transpose(1, 0, 2)
            .reshape(3 * CM, CM * CM)) / (4.0 * math.sqrt(3.0 * CM))
    bcat = (fc2_w2.reshape(CM, CM, CM).transpose(1, 0, 2)
            .reshape(CM, CM * CM)) / 16.0
    trep = jnp.asarray(np.kron(np.eye(CM, dtype=np.float32),
                               np.ones((1, CM), dtype=np.float32)) * sqrt2)
    smat = jnp.asarray(np.kron(np.ones((CM, 1), dtype=np.float32),
                               np.eye(CM, dtype=np.float32)))
    l1 = lin1_w / math.sqrt(2.0 * CM)
    l2 = lin2_w * (sqrt2 / math.sqrt(CM))

    sc_gather, sc_scatter = _sc_kernels()
    hsrc, hdst = sc_gather(hn, edge_index)

    he_new, contrib = _edge_call(he, hsrc, hdst, emb,
                                 norm.reshape(E, 1).astype(f32),
                                 w1a, w2a, acat, trep, smat, bcat)

    nf = sc_scatter(contrib, edge_index)

    hn_new = _node_call(hn, nf, l1, l2)
    return hn_new, he_new


# speedup vs baseline: 4.7015x; 1.6237x over previous
import functools
import math

import jax
import jax.numpy as jnp
import numpy as np
from jax import lax
from jax.experimental import pallas as pl
from jax.experimental.pallas import tpu as pltpu
from jax.experimental.pallas import tpu_sc as plsc

N = 10000
E = 160000
CM = 16

NC = 2
NS = 16
NW = NC * NS
EPW = E // NW
CH = 128
NFULL = EPW // CH
TAIL = EPW - NFULL * CH
RPS = N // NS

BE = 3200
GE = E // BE
PB = BE // 8
RUN = 200
NRUN = EPW // RUN
PN = N // 8


def _run_dst(a):
    q = a // PB
    return (q // 8) * PB + a % PB, 16 * (q % 8)


def _sc_gather_body(hn_hbm, ei_hbm, hsrc_hbm, hdst_hbm, idx_v, rows_v, sem):
    cid = lax.axis_index("c")
    sid = lax.axis_index("s")
    wid = sid * NC + cid
    base = wid * EPW
    for p, out_hbm in ((0, hsrc_hbm), (1, hdst_hbm)):
        pltpu.sync_copy(ei_hbm.at[p, pl.ds(base, EPW)], idx_v)

        def _group(g, carry):
            descs = []
            for b in range(3):
                o = (g * 3 + b) * CH
                descs.append(pltpu.async_copy(
                    hn_hbm.at[idx_v.at[pl.ds(o, CH)]],
                    rows_v.at[pl.ds(o, CH)], sem))
            for d in descs:
                d.wait()
            return carry

        lax.fori_loop(0, NFULL // 3, _group, 0)
        pltpu.async_copy(hn_hbm.at[idx_v.at[pl.ds(NFULL * CH, TAIL)]],
                         rows_v.at[pl.ds(NFULL * CH, TAIL)], sem).wait()
        descs = []
        for k in range(NRUN):
            r0, l0 = _run_dst(base + k * RUN)
            descs.append(pltpu.async_copy(
                rows_v.at[pl.ds(k * RUN, RUN)],
                out_hbm.at[pl.ds(r0, RUN), pl.ds(l0, CM)], sem))
        for d in descs:
            d.wait()


def _sc_scatter_body(valp_hbm, ei_hbm, out_hbm, idx2_v, idx8_v, val_v,
                     zero_v, shared, sem):
    cid = lax.axis_index("c")
    sid = lax.axis_index("s")
    wid = sid * NC + cid
    base = wid * EPW
    descs = []
    for k in range(NRUN):
        r0, l0 = _run_dst(base + k * RUN)
        descs.append(pltpu.async_copy(
            valp_hbm.at[pl.ds(r0, RUN), pl.ds(l0, CM)],
            val_v.at[pl.ds(k * RUN, RUN)], sem))
    for j in range(NFULL):
        descs.append(pltpu.async_copy(
            ei_hbm.at[1, pl.ds(base + j * CH, CH)], idx2_v.at[j], sem))
    descs.append(pltpu.async_copy(
        ei_hbm.at[1, pl.ds(base + NFULL * CH, TAIL)], idx8_v, sem))

    def _zrow(i, carry):
        zero_v[i, :] = jnp.zeros((CM,), jnp.float32)
        return carry

    lax.fori_loop(0, RPS, _zrow, 0)
    pltpu.sync_copy(zero_v, shared.at[pl.ds(sid * RPS, RPS)])
    for d in descs:
        d.wait()
    plsc.subcore_barrier()

    def _chunk(j, carry):
        pltpu.sync_copy(val_v.at[pl.ds(j * CH, CH)],
                        shared.at[idx2_v.at[j]], add=True)
        return carry

    lax.fori_loop(0, NFULL, _chunk, 0)
    pltpu.sync_copy(val_v.at[pl.ds(NFULL * CH, TAIL)],
                    shared.at[idx8_v], add=True)
    plsc.subcore_barrier()
    pltpu.sync_copy(
        shared.at[pl.ds(sid * RPS, RPS)],
        out_hbm.at[pl.ds(cid * PN + (sid % 2) * RPS, RPS),
                   pl.ds(16 * (sid // 2), CM)])


@functools.lru_cache(maxsize=None)
def _sc_kernels():
    mesh = plsc.VectorSubcoreMesh(core_axis_name="c", subcore_axis_name="s",
                                  num_cores=NC, num_subcores=NS)
    gather = pl.kernel(
        _sc_gather_body,
        out_type=(jax.ShapeDtypeStruct((E // 8, 8 * CM), jnp.float32),
                  jax.ShapeDtypeStruct((E // 8, 8 * CM), jnp.float32)),
        mesh=mesh,
        scratch_types=[
            pltpu.VMEM((EPW,), jnp.int32),
            pltpu.VMEM((EPW, CM), jnp.float32),
            pltpu.SemaphoreType.DMA,
        ],
        compiler_params=pltpu.CompilerParams(use_tc_tiling_on_sc=False),
    )
    scatter = pl.kernel(
        _sc_scatter_body,
        out_type=jax.ShapeDtypeStruct((NC * PN, 8 * CM), jnp.float32),
        mesh=mesh,
        scratch_types=[
            pltpu.VMEM((NFULL, CH), jnp.int32),
            pltpu.VMEM((TAIL,), jnp.int32),
            pltpu.VMEM((EPW, CM), jnp.float32),
            pltpu.VMEM((RPS, CM), jnp.float32),
            pltpu.VMEM_SHARED((N, CM), jnp.float32),
            pltpu.SemaphoreType.DMA,
        ],
        compiler_params=pltpu.CompilerParams(use_tc_tiling_on_sc=False),
    )
    return gather, scatter


def _unpack(xp, rows):
    return jnp.concatenate(
        [xp[:, 16 * p:16 * (p + 1)] for p in range(8)], axis=0)


def _pack(x, rows):
    pb = rows // 8
    return jnp.concatenate(
        [x[pb * p:pb * (p + 1), :] for p in range(8)], axis=1)


def _tc_edge_body(he, hsp, hdp, emb, nrm8, kr, w1a, w2a, acat, trep, smat,
                  bcat, he_new, contribp):
    h1 = jnp.maximum(jnp.dot(emb[...], w1a[...],
                             preferred_element_type=jnp.float32), 0.0)
    h2 = jnp.maximum(jnp.dot(emb[...], w2a[...],
                             preferred_element_type=jnp.float32), 0.0)
    hs = _unpack(hsp[...], BE)
    hd = _unpack(hdp[...], BE)
    feat = jnp.concatenate([he[...], hs, hd], axis=1)
    q1 = jnp.dot(feat, acat[...], preferred_element_type=jnp.float32)
    r1 = jnp.dot(h1, trep[...], preferred_element_type=jnp.float32)
    tmp = jnp.maximum(jnp.dot(q1 * r1, smat[...],
                              preferred_element_type=jnp.float32), 0.0)
    q2 = jnp.dot(tmp, bcat[...], preferred_element_type=jnp.float32)
    r2 = jnp.dot(h2, trep[...], preferred_element_type=jnp.float32)
    hen = he[...] + jnp.dot(q2 * r2, smat[...],
                            preferred_element_type=jnp.float32)
    he_new[...] = hen
    nps = jnp.dot(nrm8[...], kr[...], preferred_element_type=jnp.float32)
    contribp[...] = _pack(hen, BE) * nps


def _edge_call(he, hsp, hdp, emb, nrm8, kr, w1a, w2a, acat, trep, smat,
               bcat):
    full = lambda i: (0, 0)
    return pl.pallas_call(
        _tc_edge_body,
        grid=(GE,),
        in_specs=[
            pl.BlockSpec((BE, CM), lambda i: (i, 0)),
            pl.BlockSpec((PB, 8 * CM), lambda i: (i, 0)),
            pl.BlockSpec((PB, 8 * CM), lambda i: (i, 0)),
            pl.BlockSpec((BE, 10), lambda i: (i, 0)),
            pl.BlockSpec((PB, 8), lambda i: (i, 0)),
            pl.BlockSpec((8, 8 * CM), full),
            pl.BlockSpec((10, CM), full),
            pl.BlockSpec((10, CM), full),
            pl.BlockSpec((3 * CM, CM * CM), full),
            pl.BlockSpec((CM, CM * CM), full),
            pl.BlockSpec((CM * CM, CM), full),
            pl.BlockSpec((CM, CM * CM), full),
        ],
        out_specs=[
            pl.BlockSpec((BE, CM), lambda i: (i, 0)),
            pl.BlockSpec((PB, 8 * CM), lambda i: (i, 0)),
        ],
        out_shape=[
            jax.ShapeDtypeStruct((E, CM), jnp.float32),
            jax.ShapeDtypeStruct((E // 8, 8 * CM), jnp.float32),
        ],
        compiler_params=pltpu.CompilerParams(
            dimension_semantics=("parallel",)),
    )(he, hsp, hdp, emb, nrm8, kr, w1a, w2a, acat, trep, smat, bcat)


def _tc_node_body(hn, nfp, l1, l2, out):
    p0 = _unpack(nfp[0:PN, :], N)
    p1 = _unpack(nfp[PN:2 * PN, :], N)
    cat = jnp.concatenate([hn[...], p0 + p1], axis=1)
    h = jnp.maximum(jnp.dot(cat, l1[...],
                            preferred_element_type=jnp.float32), 0.0)
    out[...] = hn[...] + jnp.dot(h, l2[...],
                                 preferred_element_type=jnp.float32)


def _node_call(hn, nfp, l1, l2):
    return pl.pallas_call(
        _tc_node_body,
        out_shape=jax.ShapeDtypeStruct((N, CM), jnp.float32),
    )(hn, nfp, l1, l2)


def kernel(hn, he, edge_vec, emb, norm, fc1_w1, fc1_w2, fc2_w1, fc2_w2,
           lin1_w, lin2_w, edge_index):
    del edge_vec
    sqrt2 = math.sqrt(2.0)

    w1a = fc1_w1 / math.sqrt(10.0)
    w2a = fc2_w1 / math.sqrt(10.0)
    acat = (fc1_w2.reshape(CM, 3 * CM, CM).transpose(1, 0, 2)
            .reshape(3 * CM, CM * CM)) / (4.0 * math.sqrt(3.0 * CM))
    bcat = (fc2_w2.reshape(CM, CM, CM).transpose(1, 0, 2)
            .reshape(CM, CM * CM)) / 16.0
    trep = jnp.asarray(np.kron(np.eye(CM, dtype=np.float32),
                               np.ones((1, CM), dtype=np.float32)) * sqrt2)
    smat = jnp.asarray(np.kron(np.ones((CM, 1), dtype=np.float32),
                               np.eye(CM, dtype=np.float32)))
    kr = jnp.asarray(np.kron(np.eye(8, dtype=np.float32),
                             np.ones((1, CM), dtype=np.float32)))
    l1 = lin1_w / math.sqrt(2.0 * CM)
    l2 = lin2_w * (sqrt2 / math.sqrt(CM))

    nrm8 = norm.astype(jnp.float32).reshape(GE, 8, PB).transpose(0, 2, 1) \
        .reshape(E // 8, 8)

    sc_gather, sc_scatter = _sc_kernels()
    hsp, hdp = sc_gather(hn, edge_index)

    he_new, contribp = _edge_call(he, hsp, hdp, emb, nrm8, kr, w1a, w2a,
                                  acat, trep, smat, bcat)

    nfp = sc_scatter(contribp, edge_index)

    hn_new = _node_call(hn, nfp, l1, l2)
    return hn_new, he_new


# speedup vs baseline: 4.7027x; 1.0003x over previous
import functools
import math

import jax
import jax.numpy as jnp
import numpy as np
from jax import lax
from jax.experimental import pallas as pl
from jax.experimental.pallas import tpu as pltpu
from jax.experimental.pallas import tpu_sc as plsc

N = 10000
E = 160000
CM = 16

NC = 2
NS = 16
NW = NC * NS
EPW = E // NW
CH = 128
NFULL = EPW // CH
TAIL = EPW - NFULL * CH
RPS = N // NS

BE = 3200
GE = E // BE
PB = BE // 8
RUN = 200
NRUN = EPW // RUN
PN = N // 8


def _run_dst(a):
    q = a // PB
    return (q // 8) * PB + a % PB, 16 * (q % 8)


def _sc_gather_body(hn_hbm, ei_hbm, hsrc_hbm, hdst_hbm, idx_v, rows_v, sem):
    cid = lax.axis_index("c")
    sid = lax.axis_index("s")
    wid = sid * NC + cid
    base = wid * EPW
    for p, out_hbm in ((0, hsrc_hbm), (1, hdst_hbm)):
        pltpu.sync_copy(ei_hbm.at[p, pl.ds(base, EPW)], idx_v)

        def _group(g, carry):
            descs = []
            for b in range(3):
                o = (g * 3 + b) * CH
                descs.append(pltpu.async_copy(
                    hn_hbm.at[idx_v.at[pl.ds(o, CH)]],
                    rows_v.at[pl.ds(o, CH)], sem))
            for d in descs:
                d.wait()
            return carry

        lax.fori_loop(0, NFULL // 3, _group, 0)
        pltpu.async_copy(hn_hbm.at[idx_v.at[pl.ds(NFULL * CH, TAIL)]],
                         rows_v.at[pl.ds(NFULL * CH, TAIL)], sem).wait()
        descs = []
        for k in range(NRUN):
            r0, l0 = _run_dst(base + k * RUN)
            descs.append(pltpu.async_copy(
                rows_v.at[pl.ds(k * RUN, RUN)],
                out_hbm.at[pl.ds(r0, RUN), pl.ds(l0, CM)], sem))
        for d in descs:
            d.wait()


def _sc_scatter_body(valp_hbm, ei_hbm, out_hbm, idx2_v, idx8_v, val_v,
                     zero_v, shared, sem):
    cid = lax.axis_index("c")
    sid = lax.axis_index("s")
    wid = sid * NC + cid
    base = wid * EPW
    descs = []
    for k in range(NRUN):
        r0, l0 = _run_dst(base + k * RUN)
        descs.append(pltpu.async_copy(
            valp_hbm.at[pl.ds(r0, RUN), pl.ds(l0, CM)],
            val_v.at[pl.ds(k * RUN, RUN)], sem))
    for j in range(NFULL):
        descs.append(pltpu.async_copy(
            ei_hbm.at[1, pl.ds(base + j * CH, CH)], idx2_v.at[j], sem))
    descs.append(pltpu.async_copy(
        ei_hbm.at[1, pl.ds(base + NFULL * CH, TAIL)], idx8_v, sem))

    def _zrow(i, carry):
        zero_v[i, :] = jnp.zeros((CM,), jnp.float32)
        return carry

    lax.fori_loop(0, RPS, _zrow, 0)
    pltpu.sync_copy(zero_v, shared.at[pl.ds(sid * RPS, RPS)])
    for d in descs:
        d.wait()
    plsc.subcore_barrier()

    def _chunk(j, carry):
        pltpu.sync_copy(val_v.at[pl.ds(j * CH, CH)],
                        shared.at[idx2_v.at[j]], add=True)
        return carry

    lax.fori_loop(0, NFULL, _chunk, 0)
    pltpu.sync_copy(val_v.at[pl.ds(NFULL * CH, TAIL)],
                    shared.at[idx8_v], add=True)
    plsc.subcore_barrier()
    pltpu.sync_copy(
        shared.at[pl.ds(sid * RPS, RPS)],
        out_hbm.at[pl.ds(cid * PN + (sid % 2) * RPS, RPS),
                   pl.ds(16 * (sid // 2), CM)])


@functools.lru_cache(maxsize=None)
def _sc_kernels():
    mesh = plsc.VectorSubcoreMesh(core_axis_name="c", subcore_axis_name="s",
                                  num_cores=NC, num_subcores=NS)
    gather = pl.kernel(
        _sc_gather_body,
        out_type=(jax.ShapeDtypeStruct((E // 8, 8 * CM), jnp.float32),
                  jax.ShapeDtypeStruct((E // 8, 8 * CM), jnp.float32)),
        mesh=mesh,
        scratch_types=[
            pltpu.VMEM((EPW,), jnp.int32),
            pltpu.VMEM((EPW, CM), jnp.float32),
            pltpu.SemaphoreType.DMA,
        ],
        compiler_params=pltpu.CompilerParams(use_tc_tiling_on_sc=False),
    )
    scatter = pl.kernel(
        _sc_scatter_body,
        out_type=jax.ShapeDtypeStruct((NC * PN, 8 * CM), jnp.float32),
        mesh=mesh,
        scratch_types=[
            pltpu.VMEM((NFULL, CH), jnp.int32),
            pltpu.VMEM((TAIL,), jnp.int32),
            pltpu.VMEM((EPW, CM), jnp.float32),
            pltpu.VMEM((RPS, CM), jnp.float32),
            pltpu.VMEM_SHARED((N, CM), jnp.float32),
            pltpu.SemaphoreType.DMA,
        ],
        compiler_params=pltpu.CompilerParams(use_tc_tiling_on_sc=False),
    )
    return gather, scatter


def _unpack(xp, rows):
    return jnp.concatenate(
        [xp[:, 16 * p:16 * (p + 1)] for p in range(8)], axis=0)


def _pack(x, rows):
    pb = rows // 8
    return jnp.concatenate(
        [x[pb * p:pb * (p + 1), :] for p in range(8)], axis=1)


_T0 = (((0,), (0,)), ((), ()))
_T1 = (((1,), (1,)), ((), ()))


def _dg(a, b, dn):
    return jax.lax.dot_general(a, b, dimension_numbers=dn,
                               preferred_element_type=jnp.float32)


def _tc_edge_body(het, hsp, hdp, embt, nrm8, kr, eye, w1a, w2a, ahe, ahs,
                  ahd, trep, smat, bcat, he_new_t, contribp):
    h1 = jnp.maximum(_dg(embt[...], w1a[...], _T0), 0.0)
    h2 = jnp.maximum(_dg(embt[...], w2a[...], _T0), 0.0)
    hs = _unpack(hsp[...], BE)
    hd = _unpack(hdp[...], BE)
    q1 = (_dg(het[...], ahe[...], _T0)
          + jnp.dot(hs, ahs[...], preferred_element_type=jnp.float32)
          + jnp.dot(hd, ahd[...], preferred_element_type=jnp.float32))
    r1 = jnp.dot(h1, trep[...], preferred_element_type=jnp.float32)
    tmp = jnp.maximum(jnp.dot(q1 * r1, smat[...],
                              preferred_element_type=jnp.float32), 0.0)
    q2 = jnp.dot(tmp, bcat[...], preferred_element_type=jnp.float32)
    r2 = jnp.dot(h2, trep[...], preferred_element_type=jnp.float32)
    hen = _dg(het[...], eye[...], _T0) + jnp.dot(
        q2 * r2, smat[...], preferred_element_type=jnp.float32)
    he_new_t[...] = _dg(eye[...], hen, _T1)
    nps = jnp.dot(nrm8[...], kr[...], preferred_element_type=jnp.float32)
    contribp[...] = _pack(hen, BE) * nps


def _edge_call(het, hsp, hdp, embt, nrm8, kr, eye, w1a, w2a, ahe, ahs, ahd,
               trep, smat, bcat):
    full = lambda i: (0, 0)
    return pl.pallas_call(
        _tc_edge_body,
        grid=(GE,),
        in_specs=[
            pl.BlockSpec((CM, BE), lambda i: (0, i)),
            pl.BlockSpec((PB, 8 * CM), lambda i: (i, 0)),
            pl.BlockSpec((PB, 8 * CM), lambda i: (i, 0)),
            pl.BlockSpec((10, BE), lambda i: (0, i)),
            pl.BlockSpec((PB, 8), lambda i: (i, 0)),
            pl.BlockSpec((8, 8 * CM), full),
            pl.BlockSpec((CM, CM), full),
            pl.BlockSpec((10, CM), full),
            pl.BlockSpec((10, CM), full),
            pl.BlockSpec((CM, CM * CM), full),
            pl.BlockSpec((CM, CM * CM), full),
            pl.BlockSpec((CM, CM * CM), full),
            pl.BlockSpec((CM, CM * CM), full),
            pl.BlockSpec((CM * CM, CM), full),
            pl.BlockSpec((CM, CM * CM), full),
        ],
        out_specs=[
            pl.BlockSpec((CM, BE), lambda i: (0, i)),
            pl.BlockSpec((PB, 8 * CM), lambda i: (i, 0)),
        ],
        out_shape=[
            jax.ShapeDtypeStruct((CM, E), jnp.float32),
            jax.ShapeDtypeStruct((E // 8, 8 * CM), jnp.float32),
        ],
        compiler_params=pltpu.CompilerParams(
            dimension_semantics=("parallel",)),
    )(het, hsp, hdp, embt, nrm8, kr, eye, w1a, w2a, ahe, ahs, ahd, trep,
      smat, bcat)


def _tc_node_body(hnt, nfp, eye, l1, l2, out_t):
    hnb = _dg(hnt[...], eye[...], _T0)
    p0 = _unpack(nfp[0:PN, :], N)
    p1 = _unpack(nfp[PN:2 * PN, :], N)
    cat = jnp.concatenate([hnb, p0 + p1], axis=1)
    h = jnp.maximum(jnp.dot(cat, l1[...],
                            preferred_element_type=jnp.float32), 0.0)
    hnew = hnb + jnp.dot(h, l2[...], preferred_element_type=jnp.float32)
    out_t[...] = _dg(eye[...], hnew, _T1)


def _node_call(hnt, nfp, eye, l1, l2):
    return pl.pallas_call(
        _tc_node_body,
        out_shape=jax.ShapeDtypeStruct((CM, N), jnp.float32),
    )(hnt, nfp, eye, l1, l2)


def kernel(hn, he, edge_vec, emb, norm, fc1_w1, fc1_w2, fc2_w1, fc2_w2,
           lin1_w, lin2_w, edge_index):
    del edge_vec
    sqrt2 = math.sqrt(2.0)

    w1a = fc1_w1 / math.sqrt(10.0)
    w2a = fc2_w1 / math.sqrt(10.0)
    acat = (fc1_w2.reshape(CM, 3 * CM, CM).transpose(1, 0, 2)
            .reshape(3 * CM, CM * CM)) / (4.0 * math.sqrt(3.0 * CM))
    bcat = (fc2_w2.reshape(CM, CM, CM).transpose(1, 0, 2)
            .reshape(CM, CM * CM)) / 16.0
    trep = jnp.asarray(np.kron(np.eye(CM, dtype=np.float32),
                               np.ones((1, CM), dtype=np.float32)) * sqrt2)
    smat = jnp.asarray(np.kron(np.ones((CM, 1), dtype=np.float32),
                               np.eye(CM, dtype=np.float32)))
    kr = jnp.asarray(np.kron(np.eye(8, dtype=np.float32),
                             np.ones((1, CM), dtype=np.float32)))
    eye = jnp.asarray(np.eye(CM, dtype=np.float32))
    l1 = lin1_w / math.sqrt(2.0 * CM)
    l2 = lin2_w * (sqrt2 / math.sqrt(CM))
    ahe, ahs, ahd = acat[:CM], acat[CM:2 * CM], acat[2 * CM:]

    nrm8 = norm.astype(jnp.float32).reshape(GE, 8, PB).transpose(0, 2, 1) \
        .reshape(E // 8, 8)

    sc_gather, sc_scatter = _sc_kernels()
    hsp, hdp = sc_gather(hn, edge_index)

    he_new_t, contribp = _edge_call(he.T, hsp, hdp, emb.T, nrm8, kr, eye,
                                    w1a, w2a, ahe, ahs, ahd, trep, smat,
                                    bcat)

    nfp = sc_scatter(contribp, edge_index)

    hn_new_t = _node_call(hn.T, nfp, eye, l1, l2)
    return hn_new_t.T, he_new_t.T


# speedup vs baseline: 4.7050x; 1.0005x over previous
import functools
import math

import jax
import jax.numpy as jnp
import numpy as np
from jax import lax
from jax.experimental import pallas as pl
from jax.experimental.pallas import tpu as pltpu
from jax.experimental.pallas import tpu_sc as plsc

N = 10000
E = 160000
CM = 16

NC = 2
NS = 16
NW = NC * NS
EPW = E // NW
CH = 128
NFULL = EPW // CH
TAIL = EPW - NFULL * CH
RPS = N // NS

BE = 3200
GE = E // BE
PB = BE // 8
RUN = 200
NRUN = EPW // RUN
PN = N // 8


def _run_dst(a):
    q = a // PB
    return (q // 8) * PB + a % PB, 16 * (q % 8)


def _sc_gather_body(hn_hbm, ei_hbm, hsrc_hbm, hdst_hbm, idx_v, rows_v, sem):
    cid = lax.axis_index("c")
    sid = lax.axis_index("s")
    wid = sid * NC + cid
    base = wid * EPW
    for p, out_hbm in ((0, hsrc_hbm), (1, hdst_hbm)):
        pltpu.sync_copy(ei_hbm.at[p, pl.ds(base, EPW)], idx_v)

        def _group(g, carry):
            descs = []
            for b in range(3):
                o = (g * 3 + b) * CH
                descs.append(pltpu.async_copy(
                    hn_hbm.at[idx_v.at[pl.ds(o, CH)]],
                    rows_v.at[pl.ds(o, CH)], sem))
            for d in descs:
                d.wait()
            return carry

        lax.fori_loop(0, NFULL // 3, _group, 0)
        pltpu.async_copy(hn_hbm.at[idx_v.at[pl.ds(NFULL * CH, TAIL)]],
                         rows_v.at[pl.ds(NFULL * CH, TAIL)], sem).wait()
        descs = []
        for k in range(NRUN):
            r0, l0 = _run_dst(base + k * RUN)
            descs.append(pltpu.async_copy(
                rows_v.at[pl.ds(k * RUN, RUN)],
                out_hbm.at[pl.ds(r0, RUN), pl.ds(l0, CM)], sem))
        for d in descs:
            d.wait()


def _sc_scatter_body(valp_hbm, ei_hbm, out_hbm, idx2_v, idx8_v, val_v,
                     zero_v, shared, sem):
    cid = lax.axis_index("c")
    sid = lax.axis_index("s")
    wid = sid * NC + cid
    base = wid * EPW
    descs = []
    for k in range(NRUN):
        r0, l0 = _run_dst(base + k * RUN)
        descs.append(pltpu.async_copy(
            valp_hbm.at[pl.ds(r0, RUN), pl.ds(l0, CM)],
            val_v.at[pl.ds(k * RUN, RUN)], sem))
    for j in range(NFULL):
        descs.append(pltpu.async_copy(
            ei_hbm.at[1, pl.ds(base + j * CH, CH)], idx2_v.at[j], sem))
    descs.append(pltpu.async_copy(
        ei_hbm.at[1, pl.ds(base + NFULL * CH, TAIL)], idx8_v, sem))

    def _zrow(i, carry):
        zero_v[i, :] = jnp.zeros((CM,), jnp.float32)
        return carry

    lax.fori_loop(0, RPS, _zrow, 0)
    pltpu.sync_copy(zero_v, shared.at[pl.ds(sid * RPS, RPS)])
    for d in descs:
        d.wait()
    plsc.subcore_barrier()

    def _chunk(j, carry):
        pltpu.sync_copy(val_v.at[pl.ds(j * CH, CH)],
                        shared.at[idx2_v.at[j]], add=True)
        return carry

    lax.fori_loop(0, NFULL, _chunk, 0)
    pltpu.sync_copy(val_v.at[pl.ds(NFULL * CH, TAIL)],
                    shared.at[idx8_v], add=True)
    plsc.subcore_barrier()
    pltpu.sync_copy(
        shared.at[pl.ds(sid * RPS, RPS)],
        out_hbm.at[pl.ds(cid * PN + (sid % 2) * RPS, RPS),
                   pl.ds(16 * (sid // 2), CM)])


@functools.lru_cache(maxsize=None)
def _sc_kernels():
    mesh = plsc.VectorSubcoreMesh(core_axis_name="c", subcore_axis_name="s",
                                  num_cores=NC, num_subcores=NS)
    gather = pl.kernel(
        _sc_gather_body,
        out_type=(jax.ShapeDtypeStruct((E // 8, 8 * CM), jnp.float32),
                  jax.ShapeDtypeStruct((E // 8, 8 * CM), jnp.float32)),
        mesh=mesh,
        scratch_types=[
            pltpu.VMEM((EPW,), jnp.int32),
            pltpu.VMEM((EPW, CM), jnp.float32),
            pltpu.SemaphoreType.DMA,
        ],
        compiler_params=pltpu.CompilerParams(use_tc_tiling_on_sc=False),
    )
    scatter = pl.kernel(
        _sc_scatter_body,
        out_type=jax.ShapeDtypeStruct((NC * PN, 8 * CM), jnp.float32),
        mesh=mesh,
        scratch_types=[
            pltpu.VMEM((NFULL, CH), jnp.int32),
            pltpu.VMEM((TAIL,), jnp.int32),
            pltpu.VMEM((EPW, CM), jnp.float32),
            pltpu.VMEM((RPS, CM), jnp.float32),
            pltpu.VMEM_SHARED((N, CM), jnp.float32),
            pltpu.SemaphoreType.DMA,
        ],
        compiler_params=pltpu.CompilerParams(use_tc_tiling_on_sc=False),
    )
    return gather, scatter


def _unpack(xp, rows):
    return jnp.concatenate(
        [xp[:, 16 * p:16 * (p + 1)] for p in range(8)], axis=0)


def _pack(x, rows):
    pb = rows // 8
    return jnp.concatenate(
        [x[pb * p:pb * (p + 1), :] for p in range(8)], axis=1)


_T0 = (((0,), (0,)), ((), ()))
_T1 = (((1,), (1,)), ((), ()))


def _dg(a, b, dn):
    return jax.lax.dot_general(a, b, dimension_numbers=dn,
                               preferred_element_type=jnp.float32)


def _tc_edge_body(het, hsp, hdp, embt, nrm8, kr, eye, w1a, w2a, ahe, ahs,
                  ahd, trep, smat, bcat, he_new_t, contribp):
    h1 = jnp.maximum(_dg(embt[...], w1a[...], _T0), 0.0)
    h2 = jnp.maximum(_dg(embt[...], w2a[...], _T0), 0.0)
    hs = _unpack(hsp[...], BE)
    hd = _unpack(hdp[...], BE)
    q1 = (_dg(het[...], ahe[...], _T0)
          + jnp.dot(hs, ahs[...], preferred_element_type=jnp.float32)
          + jnp.dot(hd, ahd[...], preferred_element_type=jnp.float32))
    r1 = jnp.dot(h1, trep[...], preferred_element_type=jnp.float32)
    tmp = jnp.maximum(jnp.dot(q1 * r1, smat[...],
                              preferred_element_type=jnp.float32), 0.0)
    q2 = jnp.dot(tmp, bcat[...], preferred_element_type=jnp.float32)
    r2 = jnp.dot(h2, trep[...], preferred_element_type=jnp.float32)
    hen = _dg(het[...], eye[...], _T0) + jnp.dot(
        q2 * r2, smat[...], preferred_element_type=jnp.float32)
    he_new_t[...] = _dg(eye[...], hen, _T1)
    nps = jnp.dot(nrm8[...], kr[...], preferred_element_type=jnp.float32)
    contribp[...] = _pack(hen, BE) * nps


def _edge_call(het, hsp, hdp, embt, nrm8, kr, eye, w1a, w2a, ahe, ahs, ahd,
               trep, smat, bcat):
    full = lambda i: (0, 0)
    return pl.pallas_call(
        _tc_edge_body,
        grid=(GE,),
        in_specs=[
            pl.BlockSpec((CM, BE), lambda i: (0, i)),
            pl.BlockSpec((PB, 8 * CM), lambda i: (i, 0)),
            pl.BlockSpec((PB, 8 * CM), lambda i: (i, 0)),
            pl.BlockSpec((10, BE), lambda i: (0, i)),
            pl.BlockSpec((PB, 8), lambda i: (i, 0)),
            pl.BlockSpec((8, 8 * CM), full),
            pl.BlockSpec((CM, CM), full),
            pl.BlockSpec((10, CM), full),
            pl.BlockSpec((10, CM), full),
            pl.BlockSpec((CM, CM * CM), full),
            pl.BlockSpec((CM, CM * CM), full),
            pl.BlockSpec((CM, CM * CM), full),
            pl.BlockSpec((CM, CM * CM), full),
            pl.BlockSpec((CM * CM, CM), full),
            pl.BlockSpec((CM, CM * CM), full),
        ],
        out_specs=[
            pl.BlockSpec((CM, BE), lambda i: (0, i)),
            pl.BlockSpec((PB, 8 * CM), lambda i: (i, 0)),
        ],
        out_shape=[
            jax.ShapeDtypeStruct((CM, E), jnp.float32),
            jax.ShapeDtypeStruct((E // 8, 8 * CM), jnp.float32),
        ],
        compiler_params=pltpu.CompilerParams(
            dimension_semantics=("parallel",),
            fuse_transposed_lhs_in_matmul=True),
    )(het, hsp, hdp, embt, nrm8, kr, eye, w1a, w2a, ahe, ahs, ahd, trep,
      smat, bcat)


def _tc_node_body(hnt, nfp, eye, l1, l2, out_t):
    hnb = _dg(hnt[...], eye[...], _T0)
    p0 = _unpack(nfp[0:PN, :], N)
    p1 = _unpack(nfp[PN:2 * PN, :], N)
    cat = jnp.concatenate([hnb, p0 + p1], axis=1)
    h = jnp.maximum(jnp.dot(cat, l1[...],
                            preferred_element_type=jnp.float32), 0.0)
    hnew = hnb + jnp.dot(h, l2[...], preferred_element_type=jnp.float32)
    out_t[...] = _dg(eye[...], hnew, _T1)


def _node_call(hnt, nfp, eye, l1, l2):
    return pl.pallas_call(
        _tc_node_body,
        out_shape=jax.ShapeDtypeStruct((CM, N), jnp.float32),
    )(hnt, nfp, eye, l1, l2)


def kernel(hn, he, edge_vec, emb, norm, fc1_w1, fc1_w2, fc2_w1, fc2_w2,
           lin1_w, lin2_w, edge_index):
    del edge_vec
    sqrt2 = math.sqrt(2.0)

    w1a = fc1_w1 / math.sqrt(10.0)
    w2a = fc2_w1 / math.sqrt(10.0)
    acat = (fc1_w2.reshape(CM, 3 * CM, CM).transpose(1, 0, 2)
            .reshape(3 * CM, CM * CM)) / (4.0 * math.sqrt(3.0 * CM))
    bcat = (fc2_w2.reshape(CM, CM, CM).transpose(1, 0, 2)
            .reshape(CM, CM * CM)) / 16.0
    trep = jnp.asarray(np.kron(np.eye(CM, dtype=np.float32),
                               np.ones((1, CM), dtype=np.float32)) * sqrt2)
    smat = jnp.asarray(np.kron(np.ones((CM, 1), dtype=np.float32),
                               np.eye(CM, dtype=np.float32)))
    kr = jnp.asarray(np.kron(np.eye(8, dtype=np.float32),
                             np.ones((1, CM), dtype=np.float32)))
    eye = jnp.asarray(np.eye(CM, dtype=np.float32))
    l1 = lin1_w / math.sqrt(2.0 * CM)
    l2 = lin2_w * (sqrt2 / math.sqrt(CM))
    ahe, ahs, ahd = acat[:CM], acat[CM:2 * CM], acat[2 * CM:]

    nrm8 = norm.astype(jnp.float32).reshape(GE, 8, PB).transpose(0, 2, 1) \
        .reshape(E // 8, 8)

    sc_gather, sc_scatter = _sc_kernels()
    hsp, hdp = sc_gather(hn, edge_index)

    he_new_t, contribp = _edge_call(he.T, hsp, hdp, emb.T, nrm8, kr, eye,
                                    w1a, w2a, ahe, ahs, ahd, trep, smat,
                                    bcat)

    nfp = sc_scatter(contribp, edge_index)

    hn_new_t = _node_call(hn.T, nfp, eye, l1, l2)
    return hn_new_t.T, he_new_t.T


# speedup vs baseline: 4.7860x; 1.0172x over previous
import functools
import math

import jax
import jax.numpy as jnp
import numpy as np
from jax import lax
from jax.experimental import pallas as pl
from jax.experimental.pallas import tpu as pltpu
from jax.experimental.pallas import tpu_sc as plsc

N = 10000
E = 160000
CM = 16

NC = 2
NS = 16
NW = NC * NS
EPW = E // NW
CH = 128
NFULL = EPW // CH
TAIL = EPW - NFULL * CH
RPS = N // NS

BE = 6400
GE = E // BE
PB = BE // 8
RUN = 200
NRUN = EPW // RUN
PN = N // 8


def _run_dst(a):
    q = a // PB
    return (q // 8) * PB + a % PB, 16 * (q % 8)


def _sc_gather_body(hn_hbm, ei_hbm, hsrc_hbm, hdst_hbm, idx_v, rows_v, sem):
    cid = lax.axis_index("c")
    sid = lax.axis_index("s")
    wid = sid * NC + cid
    base = wid * EPW
    for p, out_hbm in ((0, hsrc_hbm), (1, hdst_hbm)):
        pltpu.sync_copy(ei_hbm.at[p, pl.ds(base, EPW)], idx_v)

        def _group(g, carry):
            descs = []
            for b in range(3):
                o = (g * 3 + b) * CH
                descs.append(pltpu.async_copy(
                    hn_hbm.at[idx_v.at[pl.ds(o, CH)]],
                    rows_v.at[pl.ds(o, CH)], sem))
            for d in descs:
                d.wait()
            return carry

        lax.fori_loop(0, NFULL // 3, _group, 0)
        pltpu.async_copy(hn_hbm.at[idx_v.at[pl.ds(NFULL * CH, TAIL)]],
                         rows_v.at[pl.ds(NFULL * CH, TAIL)], sem).wait()
        descs = []
        for k in range(NRUN):
            r0, l0 = _run_dst(base + k * RUN)
            descs.append(pltpu.async_copy(
                rows_v.at[pl.ds(k * RUN, RUN)],
                out_hbm.at[pl.ds(r0, RUN), pl.ds(l0, CM)], sem))
        for d in descs:
            d.wait()


def _sc_scatter_body(valp_hbm, ei_hbm, out_hbm, idx2_v, idx8_v, val_v,
                     zero_v, shared, sem):
    cid = lax.axis_index("c")
    sid = lax.axis_index("s")
    wid = sid * NC + cid
    base = wid * EPW
    descs = []
    for k in range(NRUN):
        r0, l0 = _run_dst(base + k * RUN)
        descs.append(pltpu.async_copy(
            valp_hbm.at[pl.ds(r0, RUN), pl.ds(l0, CM)],
            val_v.at[pl.ds(k * RUN, RUN)], sem))
    for j in range(NFULL):
        descs.append(pltpu.async_copy(
            ei_hbm.at[1, pl.ds(base + j * CH, CH)], idx2_v.at[j], sem))
    descs.append(pltpu.async_copy(
        ei_hbm.at[1, pl.ds(base + NFULL * CH, TAIL)], idx8_v, sem))

    def _zrow(i, carry):
        zero_v[i, :] = jnp.zeros((CM,), jnp.float32)
        return carry

    lax.fori_loop(0, RPS, _zrow, 0)
    pltpu.sync_copy(zero_v, shared.at[pl.ds(sid * RPS, RPS)])
    for d in descs:
        d.wait()
    plsc.subcore_barrier()

    def _chunk(j, carry):
        pltpu.sync_copy(val_v.at[pl.ds(j * CH, CH)],
                        shared.at[idx2_v.at[j]], add=True)
        return carry

    lax.fori_loop(0, NFULL, _chunk, 0)
    pltpu.sync_copy(val_v.at[pl.ds(NFULL * CH, TAIL)],
                    shared.at[idx8_v], add=True)
    plsc.subcore_barrier()
    pltpu.sync_copy(
        shared.at[pl.ds(sid * RPS, RPS)],
        out_hbm.at[pl.ds(cid * PN + (sid % 2) * RPS, RPS),
                   pl.ds(16 * (sid // 2), CM)])


@functools.lru_cache(maxsize=None)
def _sc_kernels():
    mesh = plsc.VectorSubcoreMesh(core_axis_name="c", subcore_axis_name="s",
                                  num_cores=NC, num_subcores=NS)
    gather = pl.kernel(
        _sc_gather_body,
        out_type=(jax.ShapeDtypeStruct((E // 8, 8 * CM), jnp.float32),
                  jax.ShapeDtypeStruct((E // 8, 8 * CM), jnp.float32)),
        mesh=mesh,
        scratch_types=[
            pltpu.VMEM((EPW,), jnp.int32),
            pltpu.VMEM((EPW, CM), jnp.float32),
            pltpu.SemaphoreType.DMA,
        ],
        compiler_params=pltpu.CompilerParams(use_tc_tiling_on_sc=False),
    )
    scatter = pl.kernel(
        _sc_scatter_body,
        out_type=jax.ShapeDtypeStruct((NC * PN, 8 * CM), jnp.float32),
        mesh=mesh,
        scratch_types=[
            pltpu.VMEM((NFULL, CH), jnp.int32),
            pltpu.VMEM((TAIL,), jnp.int32),
            pltpu.VMEM((EPW, CM), jnp.float32),
            pltpu.VMEM((RPS, CM), jnp.float32),
            pltpu.VMEM_SHARED((N, CM), jnp.float32),
            pltpu.SemaphoreType.DMA,
        ],
        compiler_params=pltpu.CompilerParams(use_tc_tiling_on_sc=False),
    )
    return gather, scatter


def _unpack(xp, rows):
    return jnp.concatenate(
        [xp[:, 16 * p:16 * (p + 1)] for p in range(8)], axis=0)


def _pack(x, rows):
    pb = rows // 8
    return jnp.concatenate(
        [x[pb * p:pb * (p + 1), :] for p in range(8)], axis=1)


_T0 = (((0,), (0,)), ((), ()))
_T1 = (((1,), (1,)), ((), ()))


def _dg(a, b, dn):
    return jax.lax.dot_general(a, b, dimension_numbers=dn,
                               preferred_element_type=jnp.float32)


def _tc_edge_body(het, hsp, hdp, embt, nrm8, kr, eye, w1a, w2a, ahe, ahs,
                  ahd, trep, smat, bcat, he_new_t, contribp):
    h1 = jnp.maximum(_dg(embt[...], w1a[...], _T0), 0.0)
    h2 = jnp.maximum(_dg(embt[...], w2a[...], _T0), 0.0)
    hs = _unpack(hsp[...], BE)
    hd = _unpack(hdp[...], BE)
    q1 = (_dg(het[...], ahe[...], _T0)
          + jnp.dot(hs, ahs[...], preferred_element_type=jnp.float32)
          + jnp.dot(hd, ahd[...], preferred_element_type=jnp.float32))
    r1 = jnp.dot(h1, trep[...], preferred_element_type=jnp.float32)
    tmp = jnp.maximum(jnp.dot(q1 * r1, smat[...],
                              preferred_element_type=jnp.float32), 0.0)
    q2 = jnp.dot(tmp, bcat[...], preferred_element_type=jnp.float32)
    r2 = jnp.dot(h2, trep[...], preferred_element_type=jnp.float32)
    hen = _dg(het[...], eye[...], _T0) + jnp.dot(
        q2 * r2, smat[...], preferred_element_type=jnp.float32)
    he_new_t[...] = _dg(eye[...], hen, _T1)
    nps = jnp.dot(nrm8[...], kr[...], preferred_element_type=jnp.float32)
    contribp[...] = _pack(hen, BE) * nps


def _edge_call(het, hsp, hdp, embt, nrm8, kr, eye, w1a, w2a, ahe, ahs, ahd,
               trep, smat, bcat):
    full = lambda i: (0, 0)
    return pl.pallas_call(
        _tc_edge_body,
        grid=(GE,),
        in_specs=[
            pl.BlockSpec((CM, BE), lambda i: (0, i)),
            pl.BlockSpec((PB, 8 * CM), lambda i: (i, 0)),
            pl.BlockSpec((PB, 8 * CM), lambda i: (i, 0)),
            pl.BlockSpec((10, BE), lambda i: (0, i)),
            pl.BlockSpec((PB, 8), lambda i: (i, 0)),
            pl.BlockSpec((8, 8 * CM), full),
            pl.BlockSpec((CM, CM), full),
            pl.BlockSpec((10, CM), full),
            pl.BlockSpec((10, CM), full),
            pl.BlockSpec((CM, CM * CM), full),
            pl.BlockSpec((CM, CM * CM), full),
            pl.BlockSpec((CM, CM * CM), full),
            pl.BlockSpec((CM, CM * CM), full),
            pl.BlockSpec((CM * CM, CM), full),
            pl.BlockSpec((CM, CM * CM), full),
        ],
        out_specs=[
            pl.BlockSpec((CM, BE), lambda i: (0, i)),
            pl.BlockSpec((PB, 8 * CM), lambda i: (i, 0)),
        ],
        out_shape=[
            jax.ShapeDtypeStruct((CM, E), jnp.float32),
            jax.ShapeDtypeStruct((E // 8, 8 * CM), jnp.float32),
        ],
        compiler_params=pltpu.CompilerParams(
            dimension_semantics=("parallel",),
            fuse_transposed_lhs_in_matmul=True),
    )(het, hsp, hdp, embt, nrm8, kr, eye, w1a, w2a, ahe, ahs, ahd, trep,
      smat, bcat)


def _tc_node_body(hnt, nfp, eye, l1, l2, out_t):
    hnb = _dg(hnt[...], eye[...], _T0)
    p0 = _unpack(nfp[0:PN, :], N)
    p1 = _unpack(nfp[PN:2 * PN, :], N)
    cat = jnp.concatenate([hnb, p0 + p1], axis=1)
    h = jnp.maximum(jnp.dot(cat, l1[...],
                            preferred_element_type=jnp.float32), 0.0)
    hnew = hnb + jnp.dot(h, l2[...], preferred_element_type=jnp.float32)
    out_t[...] = _dg(eye[...], hnew, _T1)


def _node_call(hnt, nfp, eye, l1, l2):
    return pl.pallas_call(
        _tc_node_body,
        out_shape=jax.ShapeDtypeStruct((CM, N), jnp.float32),
    )(hnt, nfp, eye, l1, l2)


def kernel(hn, he, edge_vec, emb, norm, fc1_w1, fc1_w2, fc2_w1, fc2_w2,
           lin1_w, lin2_w, edge_index):
    del edge_vec
    sqrt2 = math.sqrt(2.0)

    w1a = fc1_w1 / math.sqrt(10.0)
    w2a = fc2_w1 / math.sqrt(10.0)
    acat = (fc1_w2.reshape(CM, 3 * CM, CM).transpose(1, 0, 2)
            .reshape(3 * CM, CM * CM)) / (4.0 * math.sqrt(3.0 * CM))
    bcat = (fc2_w2.reshape(CM, CM, CM).transpose(1, 0, 2)
            .reshape(CM, CM * CM)) / 16.0
    trep = jnp.asarray(np.kron(np.eye(CM, dtype=np.float32),
                               np.ones((1, CM), dtype=np.float32)) * sqrt2)
    smat = jnp.asarray(np.kron(np.ones((CM, 1), dtype=np.float32),
                               np.eye(CM, dtype=np.float32)))
    kr = jnp.asarray(np.kron(np.eye(8, dtype=np.float32),
                             np.ones((1, CM), dtype=np.float32)))
    eye = jnp.asarray(np.eye(CM, dtype=np.float32))
    l1 = lin1_w / math.sqrt(2.0 * CM)
    l2 = lin2_w * (sqrt2 / math.sqrt(CM))
    ahe, ahs, ahd = acat[:CM], acat[CM:2 * CM], acat[2 * CM:]

    nrm8 = norm.astype(jnp.float32).reshape(GE, 8, PB).transpose(0, 2, 1) \
        .reshape(E // 8, 8)

    sc_gather, sc_scatter = _sc_kernels()
    hsp, hdp = sc_gather(hn, edge_index)

    he_new_t, contribp = _edge_call(he.T, hsp, hdp, emb.T, nrm8, kr, eye,
                                    w1a, w2a, ahe, ahs, ahd, trep, smat,
                                    bcat)

    nfp = sc_scatter(contribp, edge_index)

    hn_new_t = _node_call(hn.T, nfp, eye, l1, l2)
    return hn_new_t.T, he_new_t.T
